# baseline (speedup 1.0000x reference)
import functools

import jax
import jax.numpy as jnp
from jax import lax
from jax.experimental import pallas as pl
from jax.experimental.pallas import tpu as pltpu
from jax.experimental.pallas import tpu_sc as plsc

T, D, I, E, K = 2048, 768, 1024, 8, 2
D2 = D // 2
RB = 128
NSLOT = 5120
NBLK = NSLOT // RB
WREP = 128
NC, NS = 2, 16
NW = NC * NS
NEG = -1e30


def _fiota(shape, dim):
    return lax.broadcasted_iota(jnp.int32, shape, dim).astype(jnp.float32)


DCH = 128
NDC = D // DCH


def _route_body(x_ref, wg_ref, bg_ref, pos0_ref, pos1_ref, w0_ref, w1_ref,
                be_ref, rs_ref, par_ref, nxe_ref, hn_ref, nreal_ref, lac_v):
    b = pl.program_id(0)
    part = lax.dot_general(x_ref[...], wg_ref[...], (((1,), (1,)), ((), ())),
                           preferred_element_type=jnp.float32)

    @pl.when(b == 0)
    def _():
        lac_v[...] = part + bg_ref[...]

    @pl.when(b > 0)
    def _():
        lac_v[...] = lac_v[...] + part

    @pl.when(b == NDC - 1)
    def _():
        _route_tail(lac_v[...], pos0_ref, pos1_ref, w0_ref, w1_ref,
                    be_ref, rs_ref, par_ref, nxe_ref, hn_ref, nreal_ref)


def _route_tail(logits, pos0_ref, pos1_ref, w0_ref, w1_ref,
                be_ref, rs_ref, par_ref, nxe_ref, hn_ref, nreal_ref):
    m = jnp.max(logits, axis=-1, keepdims=True)
    p = jnp.exp(logits - m)
    s = p / jnp.sum(p, axis=-1, keepdims=True)

    ei = _fiota((T, E), 1)
    m0 = jnp.max(s, axis=-1, keepdims=True)
    i0 = jnp.min(jnp.where(s >= m0, ei, float(E)), axis=-1, keepdims=True)
    s2 = jnp.where(ei == i0, NEG, s)
    m1 = jnp.max(s2, axis=-1, keepdims=True)
    i1 = jnp.min(jnp.where(s2 >= m1, ei, float(E)), axis=-1, keepdims=True)

    w0_ref[...] = jnp.broadcast_to(m0, (T, WREP))
    w1_ref[...] = jnp.broadcast_to(m1, (T, WREP))

    m0e = (ei == i0).astype(jnp.float32)
    m1e = (ei == i1).astype(jnp.float32)

    ri = _fiota((T, T), 0)
    ci = _fiota((T, T), 1)
    ltri = (ci <= ri).astype(jnp.bfloat16)
    mcat = jnp.concatenate([m0e, m1e], axis=1).astype(jnp.bfloat16)
    c01 = jnp.dot(ltri, mcat, preferred_element_type=jnp.float32)
    c0 = c01[:, :E]
    c1 = c01[:, E:]

    n0 = c0[T - 1:T, :]
    n1 = c1[T - 1:T, :]
    n = n0 + n1
    nblk = jnp.floor((n + (RB - 1.0)) * (1.0 / RB))
    r8 = _fiota((E, E), 0)
    c8 = _fiota((E, E), 1)
    sutri = (r8 < c8).astype(jnp.float32)
    blkoff = jnp.dot(nblk, sutri, preferred_element_type=jnp.float32)
    off = blkoff * RB

    ones8 = jnp.ones((1, E), jnp.float32)
    dnt = (((1,), (1,)), ((), ()))

    def _posrow(mask, q):
        qh = jnp.floor(q * (1.0 / RB))
        ql = q - RB * qh
        return (RB * lax.dot_general(ones8, mask * qh, dnt,
                                     preferred_element_type=jnp.float32)
                + lax.dot_general(ones8, mask * ql, dnt,
                                  preferred_element_type=jnp.float32))

    pos0 = _posrow(m0e, off + c0 - m0e)
    pos1 = _posrow(m1e, off + n0 + c1 - m1e)
    pos0_ref[...] = pos0.astype(jnp.int32)
    pos1_ref[...] = pos1.astype(jnp.int32)

    bi = _fiota((E, NBLK), 1)
    eb = _fiota((E, NBLK), 0)
    boffc = jnp.reshape(blkoff, (E, 1))
    nblkc = jnp.reshape(nblk, (E, 1))
    ind = ((bi >= boffc) & (bi < boffc + nblkc)).astype(jnp.float32)
    tail = ((eb == (E - 1.0)) &
            (jnp.sum(ind, axis=0, keepdims=True) == 0.0)).astype(jnp.float32)
    ind2 = jnp.minimum(ind + tail, 1.0)
    bex = jnp.sum(eb * ind2, axis=0, keepdims=True)
    be_ref[...] = bex.astype(jnp.int32)

    presentc = jnp.minimum(
        jnp.sum(ind, axis=1, keepdims=True).astype(jnp.bool_).astype(
            jnp.float32)
        + (_fiota((E, 1), 0) == (E - 1.0)).astype(jnp.float32), 1.0)
    ltm = (c8 < r8).astype(jnp.float32)
    rankc = jnp.dot(ltm, presentc, preferred_element_type=jnp.float32)
    parc = rankc - 2.0 * jnp.floor(rankc * 0.5)
    gtm = (c8 > r8).astype(jnp.float32)
    prow = jnp.sum((r8 == c8).astype(jnp.float32) * presentc, axis=0,
                   keepdims=True)
    candm = gtm * prow * c8 + (1.0 - gtm * prow) * 1e9
    nxt = jnp.min(candm, axis=1, keepdims=True)
    nxt = jnp.where(nxt > float(E), _fiota((E, 1), 0), nxt)
    par_b = jnp.sum(parc * ind2, axis=0, keepdims=True)
    nxe_b = jnp.sum(nxt * ind2, axis=0, keepdims=True)
    rs_b = (bex != jnp.concatenate([bex[:, :1] - 1.0, bex[:, :NBLK - 1]],
                                   axis=1)).astype(jnp.float32)
    hn_b = (nxe_b != bex).astype(jnp.float32)
    par_ref[...] = par_b.astype(jnp.int32)
    nxe_ref[...] = nxe_b.astype(jnp.int32)
    rs_ref[...] = rs_b.astype(jnp.int32)
    hn_ref[...] = hn_b.astype(jnp.int32)
    nreal_ref[...] = jnp.broadcast_to(
        jnp.sum(nblk, axis=1, keepdims=True), (1, NBLK)).astype(jnp.int32)


def _route(x, Wg, bg):
    return pl.pallas_call(
        _route_body,
        grid=(NDC,),
        in_specs=[
            pl.BlockSpec((T, DCH), lambda b: (0, b)),
            pl.BlockSpec((E, DCH), lambda b: (0, b)),
            pl.BlockSpec((1, E), lambda b: (0, 0)),
        ],
        out_specs=(
            pl.BlockSpec((1, T), lambda b: (0, 0)),
            pl.BlockSpec((1, T), lambda b: (0, 0)),
            pl.BlockSpec((T, WREP), lambda b: (0, 0)),
            pl.BlockSpec((T, WREP), lambda b: (0, 0)),
            pl.BlockSpec((1, NBLK), lambda b: (0, 0)),
            pl.BlockSpec((1, NBLK), lambda b: (0, 0)),
            pl.BlockSpec((1, NBLK), lambda b: (0, 0)),
            pl.BlockSpec((1, NBLK), lambda b: (0, 0)),
            pl.BlockSpec((1, NBLK), lambda b: (0, 0)),
            pl.BlockSpec((1, NBLK), lambda b: (0, 0)),
        ),
        scratch_shapes=[pltpu.VMEM((T, E), jnp.float32)],
        compiler_params=pltpu.CompilerParams(
            dimension_semantics=("arbitrary",),
        ),
        out_shape=(
            jax.ShapeDtypeStruct((1, T), jnp.int32),
            jax.ShapeDtypeStruct((1, T), jnp.int32),
            jax.ShapeDtypeStruct((T, WREP), jnp.float32),
            jax.ShapeDtypeStruct((T, WREP), jnp.float32),
            jax.ShapeDtypeStruct((1, NBLK), jnp.int32),
            jax.ShapeDtypeStruct((1, NBLK), jnp.int32),
            jax.ShapeDtypeStruct((1, NBLK), jnp.int32),
            jax.ShapeDtypeStruct((1, NBLK), jnp.int32),
            jax.ShapeDtypeStruct((1, NBLK), jnp.int32),
            jax.ShapeDtypeStruct((1, NBLK), jnp.int32),
        ),
    )(x, Wg, bg)


_TPW = T // NW
_NCH = 4
_CCH = _TPW // _NCH


@functools.cache
def _sc_mesh():
    return plsc.VectorSubcoreMesh(core_axis_name="c", subcore_axis_name="s",
                                  num_cores=NC, num_subcores=NS)


@functools.cache
def _make_dispatch():
    @functools.partial(
        pl.kernel,
        out_type=(
            jax.ShapeDtypeStruct((NSLOT, D), jnp.float32),
            jax.ShapeDtypeStruct((NSLOT, WREP), jnp.float32),
        ),
        mesh=_sc_mesh(),
        scratch_types=[
            pltpu.VMEM((_TPW,), jnp.int32),
            pltpu.VMEM((_TPW,), jnp.int32),
            pltpu.VMEM((_TPW, D), jnp.float32),
            pltpu.VMEM((_TPW, WREP), jnp.float32),
            pltpu.VMEM((_TPW, WREP), jnp.float32),
            pltpu.SemaphoreType.DMA,
            pltpu.SemaphoreType.DMA,
        ],
    )
    def _dispatch(x_hbm, pos0_hbm, pos1_hbm, w0_hbm, w1_hbm, xs_hbm, ws_hbm,
                  i0_v, i1_v, rows_v, w0_v, w1_v, sem0, sem1):
        wid = lax.axis_index("s") * NC + lax.axis_index("c")
        base = wid * _TPW
        sl = pl.ds(base, _TPW)
        pltpu.sync_copy(pos0_hbm.at[sl], i0_v)
        pltpu.sync_copy(pos1_hbm.at[sl], i1_v)
        pltpu.sync_copy(x_hbm.at[sl], rows_v)
        pltpu.sync_copy(w0_hbm.at[sl], w0_v)
        pltpu.sync_copy(w1_hbm.at[sl], w1_v)
        cp0 = pltpu.async_copy(rows_v, xs_hbm.at[i0_v], sem0)
        cp1 = pltpu.async_copy(rows_v, xs_hbm.at[i1_v], sem1)
        cp0.wait()
        cp1.wait()
        cp2 = pltpu.async_copy(w0_v, ws_hbm.at[i0_v], sem0)
        cp3 = pltpu.async_copy(w1_v, ws_hbm.at[i1_v], sem1)
        cp2.wait()
        cp3.wait()

    return _dispatch


def _ffn_body(be_ref, rs_ref, par_ref, nxe_ref, hn_ref, nreal_ref,
              xs_ref, w1_hbm, b1_ref, w3_hbm, b3_ref, w2_hbm, b2_ref, ws_ref,
              out_ref, w1b_v, w3b_v, w2b_v, sem0, sem1):
    b = pl.program_id(0)
    e = be_ref[b]
    slot = par_ref[b]
    sems = [sem0, sem1]

    def cps(dste, s):
        return (
            pltpu.make_async_copy(w1_hbm.at[dste], w1b_v.at[s], sems[0]),
            pltpu.make_async_copy(w3_hbm.at[dste], w3b_v.at[s], sems[1]),
            pltpu.make_async_copy(w2_hbm.at[dste], w2b_v.at[s], sems[0]),
        )

    @pl.when(b == 0)
    def _():
        for cp in cps(e, slot):
            cp.start()

    @pl.when(rs_ref[b] == 1)
    def _():
        for cp in cps(e, slot):
            cp.wait()

        @pl.when(hn_ref[b] == 1)
        def _():
            for cp in cps(nxe_ref[b], 1 - slot):
                cp.start()

    @pl.when(b < nreal_ref[0])
    def _():
        xsb = xs_ref[...].astype(jnp.bfloat16)
        w1b = w1b_v[slot].astype(jnp.bfloat16)
        w3b = w3b_v[slot].astype(jnp.bfloat16)
        g = jnp.dot(xsb, w1b, preferred_element_type=jnp.float32) + b1_ref[0]
        u = jnp.dot(xsb, w3b, preferred_element_type=jnp.float32) + b3_ref[0]
        h = g * (1.0 / (1.0 + jnp.exp(-g))) * u
        hb = h.astype(jnp.bfloat16)
        w2b = w2b_v[slot].astype(jnp.bfloat16)
        o = jnp.dot(hb, w2b, preferred_element_type=jnp.float32) + b2_ref[0]
        out_ref[...] = o * ws_ref[:, :1]


def _ffn(be, rs, par, nxe, hn, nreal, xs, W1, b1, W3, b3, W2, b2, ws):
    grid_spec = pltpu.PrefetchScalarGridSpec(
        num_scalar_prefetch=6,
        grid=(NBLK,),
        in_specs=[
            pl.BlockSpec((RB, D), lambda b, *_: (b, 0)),
            pl.BlockSpec(memory_space=pltpu.HBM),
            pl.BlockSpec((1, 1, I), lambda b, be_ref, *_: (be_ref[b], 0, 0)),
            pl.BlockSpec(memory_space=pltpu.HBM),
            pl.BlockSpec((1, 1, I), lambda b, be_ref, *_: (be_ref[b], 0, 0)),
            pl.BlockSpec(memory_space=pltpu.HBM),
            pl.BlockSpec((1, 1, D), lambda b, be_ref, *_: (be_ref[b], 0, 0)),
            pl.BlockSpec((RB, WREP), lambda b, *_: (b, 0)),
        ],
        out_specs=pl.BlockSpec((RB, D), lambda b, *_: (b, 0)),
        scratch_shapes=[
            pltpu.VMEM((2, D, I), jnp.float32),
            pltpu.VMEM((2, D, I), jnp.float32),
            pltpu.VMEM((2, I, D), jnp.float32),
            pltpu.SemaphoreType.DMA,
            pltpu.SemaphoreType.DMA,
        ],
    )
    return pl.pallas_call(
        _ffn_body,
        grid_spec=grid_spec,
        out_shape=jax.ShapeDtypeStruct((NSLOT, D), jnp.float32),
        compiler_params=pltpu.CompilerParams(
            dimension_semantics=("arbitrary",),
        ),
    )(be, rs, par, nxe, hn, nreal, xs, W1, b1, W3, b3, W2, b2, ws)


@functools.cache
def _make_combine():
    @functools.partial(
        pl.kernel,
        out_type=jax.ShapeDtypeStruct((T, D), jnp.float32),
        mesh=_sc_mesh(),
        scratch_types=[
            pltpu.VMEM((_TPW,), jnp.int32),
            pltpu.VMEM((_TPW,), jnp.int32),
            pltpu.VMEM((_TPW, D), jnp.float32),
            pltpu.VMEM((_TPW, D), jnp.float32),
            pltpu.SemaphoreType.DMA,
            pltpu.SemaphoreType.DMA,
            pltpu.SemaphoreType.DMA,
            pltpu.SemaphoreType.DMA,
        ],
    )
    def _combine(out2_hbm, pos0_hbm, pos1_hbm, y_hbm, i0_v, i1_v, r0_v, r1_v,
                 sema0, sema1, semb0, semb1):
        wid = lax.axis_index("s") * NC + lax.axis_index("c")
        base = wid * _TPW
        pltpu.sync_copy(pos0_hbm.at[pl.ds(base, _TPW)], i0_v)
        pltpu.sync_copy(pos1_hbm.at[pl.ds(base, _TPW)], i1_v)

        def start(c):
            rs = pl.ds(c * _CCH, _CCH)
            s0, s1 = (sema0, sema1) if c % 2 == 0 else (semb0, semb1)
            cp0 = pltpu.async_copy(out2_hbm.at[i0_v.at[rs]], r0_v.at[rs], s0)
            cp1 = pltpu.async_copy(out2_hbm.at[i1_v.at[rs]], r1_v.at[rs], s1)
            return cp0, cp1

        inflight = start(0)
        for c in range(_NCH):
            nxt = start(c + 1) if c + 1 < _NCH else None
            inflight[0].wait()
            inflight[1].wait()

            def row(r, _):
                for j in range(D // 16):
                    sl = pl.ds(j * 16, 16)
                    plsc.addupdate(r0_v.at[r, sl], r1_v[r, sl])
                return _

            lax.fori_loop(c * _CCH, (c + 1) * _CCH, row, 0)
            inflight = nxt
        pltpu.sync_copy(r0_v, y_hbm.at[pl.ds(base, _TPW)])

    return _combine


def kernel(x, Wg, bg, W1, b1, W3, b3, W2, b2):
    (pos0, pos1, w0, w1, be, rs, par, nxe, hn, nreal) = _route(
        x, Wg.T, jnp.reshape(bg, (1, E)))
    xs, ws = _make_dispatch()(
        x, jnp.reshape(pos0, (T,)), jnp.reshape(pos1, (T,)), w0, w1)
    out2 = _ffn(jnp.reshape(be, (NBLK,)), jnp.reshape(rs, (NBLK,)),
                jnp.reshape(par, (NBLK,)), jnp.reshape(nxe, (NBLK,)),
                jnp.reshape(hn, (NBLK,)), jnp.reshape(nreal, (NBLK,)), xs,
                W1, jnp.reshape(b1, (E, 1, I)),
                W3, jnp.reshape(b3, (E, 1, I)),
                W2, jnp.reshape(b2, (E, 1, D)), ws)
    y = _make_combine()(out2, jnp.reshape(pos0, (T,)), jnp.reshape(pos1, (T,)))
    return y


# speedup vs baseline: 1.0187x; 1.0187x over previous
import functools

import jax
import jax.numpy as jnp
from jax import lax
from jax.experimental import pallas as pl
from jax.experimental.pallas import tpu as pltpu
from jax.experimental.pallas import tpu_sc as plsc

T, D, I, E, K = 2048, 768, 1024, 8, 2
D2 = D // 2
RB = 128
NSLOT = 5120
NBLK = NSLOT // RB
WREP = 128
NC, NS = 2, 16
NW = NC * NS
NEG = -1e30


def _fiota(shape, dim):
    return lax.broadcasted_iota(jnp.int32, shape, dim).astype(jnp.float32)


def _route_body(x_ref, wg_ref, bg_ref, pos0_ref, pos1_ref, w0_ref, w1_ref,
                be_ref, rs_ref, par_ref, nxe_ref, hn_ref, nreal_ref):
    logits = lax.dot_general(x_ref[...], wg_ref[...],
                             (((1,), (1,)), ((), ())),
                             preferred_element_type=jnp.float32)
    logits = logits + bg_ref[...]
    m = jnp.max(logits, axis=-1, keepdims=True)
    p = jnp.exp(logits - m)
    s = p / jnp.sum(p, axis=-1, keepdims=True)

    ei = _fiota((T, E), 1)
    m0 = jnp.max(s, axis=-1, keepdims=True)
    i0 = jnp.min(jnp.where(s >= m0, ei, float(E)), axis=-1, keepdims=True)
    s2 = jnp.where(ei == i0, NEG, s)
    m1 = jnp.max(s2, axis=-1, keepdims=True)
    i1 = jnp.min(jnp.where(s2 >= m1, ei, float(E)), axis=-1, keepdims=True)

    w0_ref[...] = jnp.broadcast_to(m0, (T, WREP))
    w1_ref[...] = jnp.broadcast_to(m1, (T, WREP))

    m0e = (ei == i0).astype(jnp.float32)
    m1e = (ei == i1).astype(jnp.float32)

    ri = _fiota((T, T), 0)
    ci = _fiota((T, T), 1)
    ltri = (ci <= ri).astype(jnp.bfloat16)
    mcat = jnp.concatenate([m0e, m1e], axis=1).astype(jnp.bfloat16)
    c01 = jnp.dot(ltri, mcat, preferred_element_type=jnp.float32)
    c0 = c01[:, :E]
    c1 = c01[:, E:]

    n0 = c0[T - 1:T, :]
    n1 = c1[T - 1:T, :]
    n = n0 + n1
    nblk = jnp.floor((n + (RB - 1.0)) * (1.0 / RB))
    r8 = _fiota((E, E), 0)
    c8 = _fiota((E, E), 1)
    sutri = (r8 < c8).astype(jnp.float32)
    blkoff = jnp.dot(nblk, sutri, preferred_element_type=jnp.float32)
    off = blkoff * RB

    ones8 = jnp.ones((1, E), jnp.float32)
    dnt = (((1,), (1,)), ((), ()))

    def _posrow(mask, q):
        qh = jnp.floor(q * (1.0 / RB))
        ql = q - RB * qh
        return (RB * lax.dot_general(ones8, mask * qh, dnt,
                                     preferred_element_type=jnp.float32)
                + lax.dot_general(ones8, mask * ql, dnt,
                                  preferred_element_type=jnp.float32))

    pos0 = _posrow(m0e, off + c0 - m0e)
    pos1 = _posrow(m1e, off + n0 + c1 - m1e)
    pos0_ref[...] = pos0.astype(jnp.int32)
    pos1_ref[...] = pos1.astype(jnp.int32)

    bi = _fiota((E, NBLK), 1)
    eb = _fiota((E, NBLK), 0)
    boffc = jnp.reshape(blkoff, (E, 1))
    nblkc = jnp.reshape(nblk, (E, 1))
    ind = ((bi >= boffc) & (bi < boffc + nblkc)).astype(jnp.float32)
    tail = ((eb == (E - 1.0)) &
            (jnp.sum(ind, axis=0, keepdims=True) == 0.0)).astype(jnp.float32)
    ind2 = jnp.minimum(ind + tail, 1.0)
    bex = jnp.sum(eb * ind2, axis=0, keepdims=True)
    be_ref[...] = bex.astype(jnp.int32)

    presentc = jnp.minimum(
        jnp.sum(ind, axis=1, keepdims=True).astype(jnp.bool_).astype(
            jnp.float32)
        + (_fiota((E, 1), 0) == (E - 1.0)).astype(jnp.float32), 1.0)
    ltm = (c8 < r8).astype(jnp.float32)
    rankc = jnp.dot(ltm, presentc, preferred_element_type=jnp.float32)
    parc = rankc - 2.0 * jnp.floor(rankc * 0.5)
    gtm = (c8 > r8).astype(jnp.float32)
    prow = jnp.sum((r8 == c8).astype(jnp.float32) * presentc, axis=0,
                   keepdims=True)
    candm = gtm * prow * c8 + (1.0 - gtm * prow) * 1e9
    nxt = jnp.min(candm, axis=1, keepdims=True)
    nxt = jnp.where(nxt > float(E), _fiota((E, 1), 0), nxt)
    par_b = jnp.sum(parc * ind2, axis=0, keepdims=True)
    nxe_b = jnp.sum(nxt * ind2, axis=0, keepdims=True)
    rs_b = (bex != jnp.concatenate([bex[:, :1] - 1.0, bex[:, :NBLK - 1]],
                                   axis=1)).astype(jnp.float32)
    hn_b = (nxe_b != bex).astype(jnp.float32)
    par_ref[...] = par_b.astype(jnp.int32)
    nxe_ref[...] = nxe_b.astype(jnp.int32)
    rs_ref[...] = rs_b.astype(jnp.int32)
    hn_ref[...] = hn_b.astype(jnp.int32)
    nreal_ref[...] = jnp.broadcast_to(
        jnp.sum(nblk, axis=1, keepdims=True), (1, NBLK)).astype(jnp.int32)


def _route(x, Wg, bg):
    return pl.pallas_call(
        _route_body,
        out_shape=(
            jax.ShapeDtypeStruct((1, T), jnp.int32),
            jax.ShapeDtypeStruct((1, T), jnp.int32),
            jax.ShapeDtypeStruct((T, WREP), jnp.float32),
            jax.ShapeDtypeStruct((T, WREP), jnp.float32),
            jax.ShapeDtypeStruct((1, NBLK), jnp.int32),
            jax.ShapeDtypeStruct((1, NBLK), jnp.int32),
            jax.ShapeDtypeStruct((1, NBLK), jnp.int32),
            jax.ShapeDtypeStruct((1, NBLK), jnp.int32),
            jax.ShapeDtypeStruct((1, NBLK), jnp.int32),
            jax.ShapeDtypeStruct((1, NBLK), jnp.int32),
        ),
    )(x, Wg, bg)


_TPW = T // NW
_NCH = 4
_CCH = _TPW // _NCH


@functools.cache
def _sc_mesh():
    return plsc.VectorSubcoreMesh(core_axis_name="c", subcore_axis_name="s",
                                  num_cores=NC, num_subcores=NS)


@functools.cache
def _make_dispatch():
    @functools.partial(
        pl.kernel,
        out_type=(
            jax.ShapeDtypeStruct((NSLOT, D), jnp.float32),
            jax.ShapeDtypeStruct((NSLOT, WREP), jnp.float32),
        ),
        mesh=_sc_mesh(),
        scratch_types=[
            pltpu.VMEM((_TPW,), jnp.int32),
            pltpu.VMEM((_TPW,), jnp.int32),
            pltpu.VMEM((_TPW, D), jnp.float32),
            pltpu.VMEM((_TPW, WREP), jnp.float32),
            pltpu.VMEM((_TPW, WREP), jnp.float32),
            pltpu.SemaphoreType.DMA,
            pltpu.SemaphoreType.DMA,
        ],
    )
    def _dispatch(x_hbm, pos0_hbm, pos1_hbm, w0_hbm, w1_hbm, xs_hbm, ws_hbm,
                  i0_v, i1_v, rows_v, w0_v, w1_v, sem0, sem1):
        wid = lax.axis_index("s") * NC + lax.axis_index("c")
        base = wid * _TPW
        sl = pl.ds(base, _TPW)
        pltpu.sync_copy(pos0_hbm.at[sl], i0_v)
        pltpu.sync_copy(pos1_hbm.at[sl], i1_v)
        pltpu.sync_copy(x_hbm.at[sl], rows_v)
        pltpu.sync_copy(w0_hbm.at[sl], w0_v)
        pltpu.sync_copy(w1_hbm.at[sl], w1_v)
        cp0 = pltpu.async_copy(rows_v, xs_hbm.at[i0_v], sem0)
        cp1 = pltpu.async_copy(rows_v, xs_hbm.at[i1_v], sem1)
        cp0.wait()
        cp1.wait()
        cp2 = pltpu.async_copy(w0_v, ws_hbm.at[i0_v], sem0)
        cp3 = pltpu.async_copy(w1_v, ws_hbm.at[i1_v], sem1)
        cp2.wait()
        cp3.wait()

    return _dispatch


def _ffn_body(be_ref, rs_ref, par_ref, nxe_ref, hn_ref, nreal_ref,
              xs_ref, w1_hbm, b1_ref, w3_hbm, b3_ref, w2_hbm, b2_ref, ws_ref,
              out_ref, w1b_v, w3b_v, w2b_v, sem0, sem1):
    b = pl.program_id(0)
    e = be_ref[b]
    slot = par_ref[b]
    sems = [sem0, sem1]

    def cps(dste, s):
        sem = sems[0] if isinstance(s, int) and s == 0 else sems[1]
        return (
            pltpu.make_async_copy(w1_hbm.at[dste], w1b_v.at[s], sem),
            pltpu.make_async_copy(w3_hbm.at[dste], w3b_v.at[s], sem),
            pltpu.make_async_copy(w2_hbm.at[dste], w2b_v.at[s], sem),
        )

    @pl.when(b == 0)
    def _():
        for cp in cps(e, 0):
            cp.start()

    @pl.when(rs_ref[b] == 1)
    def _():
        @pl.when(hn_ref[b] == 1)
        def _():
            @pl.when(slot == 0)
            def _():
                for cp in cps(nxe_ref[b], 1):
                    cp.start()

            @pl.when(slot == 1)
            def _():
                for cp in cps(nxe_ref[b], 0):
                    cp.start()

        @pl.when(slot == 0)
        def _():
            for cp in cps(e, 0):
                cp.wait()

        @pl.when(slot == 1)
        def _():
            for cp in cps(e, 1):
                cp.wait()

    @pl.when(b < nreal_ref[0])
    def _():
        xsb = xs_ref[...].astype(jnp.bfloat16)
        w1b = w1b_v[slot].astype(jnp.bfloat16)
        w3b = w3b_v[slot].astype(jnp.bfloat16)
        g = jnp.dot(xsb, w1b, preferred_element_type=jnp.float32) + b1_ref[0]
        u = jnp.dot(xsb, w3b, preferred_element_type=jnp.float32) + b3_ref[0]
        h = g * (1.0 / (1.0 + jnp.exp(-g))) * u
        hb = h.astype(jnp.bfloat16)
        w2b = w2b_v[slot].astype(jnp.bfloat16)
        o = jnp.dot(hb, w2b, preferred_element_type=jnp.float32) + b2_ref[0]
        out_ref[...] = o * ws_ref[:, :1]


def _ffn(be, rs, par, nxe, hn, nreal, xs, W1, b1, W3, b3, W2, b2, ws):
    grid_spec = pltpu.PrefetchScalarGridSpec(
        num_scalar_prefetch=6,
        grid=(NBLK,),
        in_specs=[
            pl.BlockSpec((RB, D), lambda b, *_: (b, 0)),
            pl.BlockSpec(memory_space=pltpu.HBM),
            pl.BlockSpec((1, 1, I), lambda b, be_ref, *_: (be_ref[b], 0, 0)),
            pl.BlockSpec(memory_space=pltpu.HBM),
            pl.BlockSpec((1, 1, I), lambda b, be_ref, *_: (be_ref[b], 0, 0)),
            pl.BlockSpec(memory_space=pltpu.HBM),
            pl.BlockSpec((1, 1, D), lambda b, be_ref, *_: (be_ref[b], 0, 0)),
            pl.BlockSpec((RB, WREP), lambda b, *_: (b, 0)),
        ],
        out_specs=pl.BlockSpec((RB, D), lambda b, *_: (b, 0)),
        scratch_shapes=[
            pltpu.VMEM((2, D, I), jnp.float32),
            pltpu.VMEM((2, D, I), jnp.float32),
            pltpu.VMEM((2, I, D), jnp.float32),
            pltpu.SemaphoreType.DMA,
            pltpu.SemaphoreType.DMA,
        ],
    )
    return pl.pallas_call(
        _ffn_body,
        grid_spec=grid_spec,
        out_shape=jax.ShapeDtypeStruct((NSLOT, D), jnp.float32),
        compiler_params=pltpu.CompilerParams(
            dimension_semantics=("arbitrary",),
        ),
    )(be, rs, par, nxe, hn, nreal, xs, W1, b1, W3, b3, W2, b2, ws)


@functools.cache
def _make_combine():
    @functools.partial(
        pl.kernel,
        out_type=jax.ShapeDtypeStruct((T, D), jnp.float32),
        mesh=_sc_mesh(),
        scratch_types=[
            pltpu.VMEM((_TPW,), jnp.int32),
            pltpu.VMEM((_TPW,), jnp.int32),
            pltpu.VMEM((_TPW, D), jnp.float32),
            pltpu.VMEM((_TPW, D), jnp.float32),
            pltpu.SemaphoreType.DMA,
            pltpu.SemaphoreType.DMA,
            pltpu.SemaphoreType.DMA,
            pltpu.SemaphoreType.DMA,
        ],
    )
    def _combine(out2_hbm, pos0_hbm, pos1_hbm, y_hbm, i0_v, i1_v, r0_v, r1_v,
                 sema0, sema1, semb0, semb1):
        wid = lax.axis_index("s") * NC + lax.axis_index("c")
        base = wid * _TPW
        pltpu.sync_copy(pos0_hbm.at[pl.ds(base, _TPW)], i0_v)
        pltpu.sync_copy(pos1_hbm.at[pl.ds(base, _TPW)], i1_v)

        def start(c):
            rs = pl.ds(c * _CCH, _CCH)
            s0, s1 = (sema0, sema1) if c % 2 == 0 else (semb0, semb1)
            cp0 = pltpu.async_copy(out2_hbm.at[i0_v.at[rs]], r0_v.at[rs], s0)
            cp1 = pltpu.async_copy(out2_hbm.at[i1_v.at[rs]], r1_v.at[rs], s1)
            return cp0, cp1

        inflight = start(0)
        for c in range(_NCH):
            nxt = start(c + 1) if c + 1 < _NCH else None
            inflight[0].wait()
            inflight[1].wait()

            def row(r, _):
                for j in range(D // 16):
                    sl = pl.ds(j * 16, 16)
                    plsc.addupdate(r0_v.at[r, sl], r1_v[r, sl])
                return _

            lax.fori_loop(c * _CCH, (c + 1) * _CCH, row, 0)
            inflight = nxt
        pltpu.sync_copy(r0_v, y_hbm.at[pl.ds(base, _TPW)])

    return _combine


def kernel(x, Wg, bg, W1, b1, W3, b3, W2, b2):
    (pos0, pos1, w0, w1, be, rs, par, nxe, hn, nreal) = _route(
        x, Wg.T, jnp.reshape(bg, (1, E)))
    xs, ws = _make_dispatch()(
        x, jnp.reshape(pos0, (T,)), jnp.reshape(pos1, (T,)), w0, w1)
    out2 = _ffn(jnp.reshape(be, (NBLK,)), jnp.reshape(rs, (NBLK,)),
                jnp.reshape(par, (NBLK,)), jnp.reshape(nxe, (NBLK,)),
                jnp.reshape(hn, (NBLK,)), jnp.reshape(nreal, (NBLK,)), xs,
                W1, jnp.reshape(b1, (E, 1, I)),
                W3, jnp.reshape(b3, (E, 1, I)),
                W2, jnp.reshape(b2, (E, 1, D)), ws)
    y = _make_combine()(out2, jnp.reshape(pos0, (T,)), jnp.reshape(pos1, (T,)))
    return y


# speedup vs baseline: 1.0410x; 1.0219x over previous
import functools

import jax
import jax.numpy as jnp
from jax import lax
from jax.experimental import pallas as pl
from jax.experimental.pallas import tpu as pltpu
from jax.experimental.pallas import tpu_sc as plsc

T, D, I, E, K = 2048, 768, 1024, 8, 2
D2 = D // 2
RB = 128
NSLOT = 5120
NBLK = NSLOT // RB
WREP = 128
NC, NS = 2, 16
NW = NC * NS
NEG = -1e30


def _fiota(shape, dim):
    return lax.broadcasted_iota(jnp.int32, shape, dim).astype(jnp.float32)


def _route_body(x_ref, wg_ref, bg_ref, pos0_ref, pos1_ref, w0_ref, w1_ref,
                be_ref, rs_ref, par_ref, nxe_ref, hn_ref, nreal_ref):
    logits = lax.dot_general(x_ref[...], wg_ref[...],
                             (((1,), (1,)), ((), ())),
                             preferred_element_type=jnp.float32)
    logits = logits + bg_ref[...]
    m = jnp.max(logits, axis=-1, keepdims=True)
    p = jnp.exp(logits - m)
    s = p / jnp.sum(p, axis=-1, keepdims=True)

    ei = _fiota((T, E), 1)
    m0 = jnp.max(s, axis=-1, keepdims=True)
    i0 = jnp.min(jnp.where(s >= m0, ei, float(E)), axis=-1, keepdims=True)
    s2 = jnp.where(ei == i0, NEG, s)
    m1 = jnp.max(s2, axis=-1, keepdims=True)
    i1 = jnp.min(jnp.where(s2 >= m1, ei, float(E)), axis=-1, keepdims=True)

    w0_ref[...] = jnp.broadcast_to(m0, (T, WREP))
    w1_ref[...] = jnp.broadcast_to(m1, (T, WREP))

    m0e = (ei == i0).astype(jnp.float32)
    m1e = (ei == i1).astype(jnp.float32)

    ri = _fiota((T, T), 0)
    ci = _fiota((T, T), 1)
    ltri = (ci <= ri).astype(jnp.bfloat16)
    mcat = jnp.concatenate([m0e, m1e], axis=1).astype(jnp.bfloat16)
    c01 = jnp.dot(ltri, mcat, preferred_element_type=jnp.float32)
    c0 = c01[:, :E]
    c1 = c01[:, E:]

    n0 = c0[T - 1:T, :]
    n1 = c1[T - 1:T, :]
    n = n0 + n1
    nblk = jnp.floor((n + (RB - 1.0)) * (1.0 / RB))
    r8 = _fiota((E, E), 0)
    c8 = _fiota((E, E), 1)
    sutri = (r8 < c8).astype(jnp.float32)
    blkoff = jnp.dot(nblk, sutri, preferred_element_type=jnp.float32)
    off = blkoff * RB

    ones8 = jnp.ones((1, E), jnp.float32)
    dnt = (((1,), (1,)), ((), ()))

    def _posrow(mask, q):
        qh = jnp.floor(q * (1.0 / RB))
        ql = q - RB * qh
        return (RB * lax.dot_general(ones8, mask * qh, dnt,
                                     preferred_element_type=jnp.float32)
                + lax.dot_general(ones8, mask * ql, dnt,
                                  preferred_element_type=jnp.float32))

    pos0 = _posrow(m0e, off + c0 - m0e)
    pos1 = _posrow(m1e, off + n0 + c1 - m1e)
    pos0_ref[...] = pos0.astype(jnp.int32)
    pos1_ref[...] = pos1.astype(jnp.int32)

    bi = _fiota((E, NBLK), 1)
    eb = _fiota((E, NBLK), 0)
    boffc = jnp.reshape(blkoff, (E, 1))
    nblkc = jnp.reshape(nblk, (E, 1))
    ind = ((bi >= boffc) & (bi < boffc + nblkc)).astype(jnp.float32)
    tail = ((eb == (E - 1.0)) &
            (jnp.sum(ind, axis=0, keepdims=True) == 0.0)).astype(jnp.float32)
    ind2 = jnp.minimum(ind + tail, 1.0)
    bex = jnp.sum(eb * ind2, axis=0, keepdims=True)
    be_ref[...] = bex.astype(jnp.int32)

    presentc = jnp.minimum(
        jnp.sum(ind, axis=1, keepdims=True).astype(jnp.bool_).astype(
            jnp.float32)
        + (_fiota((E, 1), 0) == (E - 1.0)).astype(jnp.float32), 1.0)
    ltm = (c8 < r8).astype(jnp.float32)
    rankc = jnp.dot(ltm, presentc, preferred_element_type=jnp.float32)
    parc = rankc - 2.0 * jnp.floor(rankc * 0.5)
    gtm = (c8 > r8).astype(jnp.float32)
    prow = jnp.sum((r8 == c8).astype(jnp.float32) * presentc, axis=0,
                   keepdims=True)
    candm = gtm * prow * c8 + (1.0 - gtm * prow) * 1e9
    nxt = jnp.min(candm, axis=1, keepdims=True)
    nxt = jnp.where(nxt > float(E), _fiota((E, 1), 0), nxt)
    par_b = jnp.sum(parc * ind2, axis=0, keepdims=True)
    nxe_b = jnp.sum(nxt * ind2, axis=0, keepdims=True)
    rs_b = (bex != jnp.concatenate([bex[:, :1] - 1.0, bex[:, :NBLK - 1]],
                                   axis=1)).astype(jnp.float32)
    hn_b = (nxe_b != bex).astype(jnp.float32)
    par_ref[...] = par_b.astype(jnp.int32)
    nxe_ref[...] = nxe_b.astype(jnp.int32)
    rs_ref[...] = rs_b.astype(jnp.int32)
    hn_ref[...] = hn_b.astype(jnp.int32)
    nreal_ref[...] = jnp.broadcast_to(
        jnp.sum(nblk, axis=1, keepdims=True), (1, NBLK)).astype(jnp.int32)


def _route(x, Wg, bg):
    return pl.pallas_call(
        _route_body,
        out_shape=(
            jax.ShapeDtypeStruct((1, T), jnp.int32),
            jax.ShapeDtypeStruct((1, T), jnp.int32),
            jax.ShapeDtypeStruct((T, WREP), jnp.float32),
            jax.ShapeDtypeStruct((T, WREP), jnp.float32),
            jax.ShapeDtypeStruct((1, NBLK), jnp.int32),
            jax.ShapeDtypeStruct((1, NBLK), jnp.int32),
            jax.ShapeDtypeStruct((1, NBLK), jnp.int32),
            jax.ShapeDtypeStruct((1, NBLK), jnp.int32),
            jax.ShapeDtypeStruct((1, NBLK), jnp.int32),
            jax.ShapeDtypeStruct((1, NBLK), jnp.int32),
        ),
    )(x, Wg, bg)


_TPW = T // NW
_NCH = 4
_CCH = _TPW // _NCH


@functools.cache
def _sc_mesh():
    return plsc.VectorSubcoreMesh(core_axis_name="c", subcore_axis_name="s",
                                  num_cores=NC, num_subcores=NS)


@functools.cache
def _make_dispatch():
    @functools.partial(
        pl.kernel,
        out_type=(
            jax.ShapeDtypeStruct((NSLOT, D), jnp.float32),
            jax.ShapeDtypeStruct((NSLOT, WREP), jnp.float32),
        ),
        mesh=_sc_mesh(),
        scratch_types=[
            pltpu.VMEM((_TPW,), jnp.int32),
            pltpu.VMEM((_TPW,), jnp.int32),
            pltpu.VMEM((_TPW, D), jnp.float32),
            pltpu.VMEM((_TPW, WREP), jnp.float32),
            pltpu.VMEM((_TPW, WREP), jnp.float32),
            pltpu.SemaphoreType.DMA,
            pltpu.SemaphoreType.DMA,
            pltpu.SemaphoreType.DMA,
            pltpu.SemaphoreType.DMA,
            pltpu.SemaphoreType.DMA,
        ],
    )
    def _dispatch(x_hbm, pos0_hbm, pos1_hbm, w0_hbm, w1_hbm, xs_hbm, ws_hbm,
                  i0_v, i1_v, rows_v, w0_v, w1_v, s1, s2, s3, s4, s5):
        wid = lax.axis_index("s") * NC + lax.axis_index("c")
        base = wid * _TPW
        sl = pl.ds(base, _TPW)
        ci0 = pltpu.async_copy(pos0_hbm.at[sl], i0_v, s1)
        ci1 = pltpu.async_copy(pos1_hbm.at[sl], i1_v, s2)
        cx = pltpu.async_copy(x_hbm.at[sl], rows_v, s3)
        cw0 = pltpu.async_copy(w0_hbm.at[sl], w0_v, s4)
        cw1 = pltpu.async_copy(w1_hbm.at[sl], w1_v, s5)
        ci0.wait()
        cx.wait()
        sc0 = pltpu.async_copy(rows_v, xs_hbm.at[i0_v], s1)
        ci1.wait()
        sc1 = pltpu.async_copy(rows_v, xs_hbm.at[i1_v], s2)
        cw0.wait()
        cw1.wait()
        sw0 = pltpu.async_copy(w0_v, ws_hbm.at[i0_v], s3)
        sw1 = pltpu.async_copy(w1_v, ws_hbm.at[i1_v], s4)
        sc0.wait()
        sc1.wait()
        sw0.wait()
        sw1.wait()

    return _dispatch


def _ffn_body(be_ref, rs_ref, par_ref, nxe_ref, hn_ref, nreal_ref,
              xs_ref, w1_hbm, b1_ref, w3_hbm, b3_ref, w2_hbm, b2_ref, ws_ref,
              out_ref, w1b_v, w3b_v, w2b_v, sem0, sem1):
    b = pl.program_id(0)
    e = be_ref[b]
    slot = par_ref[b]
    sems = [sem0, sem1]

    def cps(dste, s):
        sem = sems[0] if isinstance(s, int) and s == 0 else sems[1]
        return (
            pltpu.make_async_copy(w1_hbm.at[dste], w1b_v.at[s], sem),
            pltpu.make_async_copy(w3_hbm.at[dste], w3b_v.at[s], sem),
            pltpu.make_async_copy(w2_hbm.at[dste], w2b_v.at[s], sem),
        )

    @pl.when(b == 0)
    def _():
        for cp in cps(e, 0):
            cp.start()

    @pl.when(rs_ref[b] == 1)
    def _():
        @pl.when(hn_ref[b] == 1)
        def _():
            @pl.when(slot == 0)
            def _():
                for cp in cps(nxe_ref[b], 1):
                    cp.start()

            @pl.when(slot == 1)
            def _():
                for cp in cps(nxe_ref[b], 0):
                    cp.start()

        @pl.when(slot == 0)
        def _():
            for cp in cps(e, 0):
                cp.wait()

        @pl.when(slot == 1)
        def _():
            for cp in cps(e, 1):
                cp.wait()

    @pl.when(b < nreal_ref[0])
    def _():
        xsb = xs_ref[...].astype(jnp.bfloat16)
        w1b = w1b_v[slot].astype(jnp.bfloat16)
        w3b = w3b_v[slot].astype(jnp.bfloat16)
        g = jnp.dot(xsb, w1b, preferred_element_type=jnp.float32) + b1_ref[0]
        u = jnp.dot(xsb, w3b, preferred_element_type=jnp.float32) + b3_ref[0]
        h = g * (1.0 / (1.0 + jnp.exp(-g))) * u
        hb = h.astype(jnp.bfloat16)
        w2b = w2b_v[slot].astype(jnp.bfloat16)
        o = jnp.dot(hb, w2b, preferred_element_type=jnp.float32) + b2_ref[0]
        out_ref[...] = o * ws_ref[:, :1]


def _ffn(be, rs, par, nxe, hn, nreal, xs, W1, b1, W3, b3, W2, b2, ws):
    grid_spec = pltpu.PrefetchScalarGridSpec(
        num_scalar_prefetch=6,
        grid=(NBLK,),
        in_specs=[
            pl.BlockSpec((RB, D), lambda b, *_: (b, 0)),
            pl.BlockSpec(memory_space=pltpu.HBM),
            pl.BlockSpec((1, 1, I), lambda b, be_ref, *_: (be_ref[b], 0, 0)),
            pl.BlockSpec(memory_space=pltpu.HBM),
            pl.BlockSpec((1, 1, I), lambda b, be_ref, *_: (be_ref[b], 0, 0)),
            pl.BlockSpec(memory_space=pltpu.HBM),
            pl.BlockSpec((1, 1, D), lambda b, be_ref, *_: (be_ref[b], 0, 0)),
            pl.BlockSpec((RB, WREP), lambda b, *_: (b, 0)),
        ],
        out_specs=pl.BlockSpec((RB, D), lambda b, *_: (b, 0)),
        scratch_shapes=[
            pltpu.VMEM((2, D, I), jnp.float32),
            pltpu.VMEM((2, D, I), jnp.float32),
            pltpu.VMEM((2, I, D), jnp.float32),
            pltpu.SemaphoreType.DMA,
            pltpu.SemaphoreType.DMA,
        ],
    )
    return pl.pallas_call(
        _ffn_body,
        grid_spec=grid_spec,
        out_shape=jax.ShapeDtypeStruct((NSLOT, D), jnp.float32),
        compiler_params=pltpu.CompilerParams(
            dimension_semantics=("arbitrary",),
        ),
    )(be, rs, par, nxe, hn, nreal, xs, W1, b1, W3, b3, W2, b2, ws)


@functools.cache
def _make_combine():
    @functools.partial(
        pl.kernel,
        out_type=jax.ShapeDtypeStruct((T, D), jnp.float32),
        mesh=_sc_mesh(),
        scratch_types=[
            pltpu.VMEM((_TPW,), jnp.int32),
            pltpu.VMEM((_TPW,), jnp.int32),
            pltpu.VMEM((_TPW, D), jnp.float32),
            pltpu.VMEM((_TPW, D), jnp.float32),
            pltpu.SemaphoreType.DMA,
            pltpu.SemaphoreType.DMA,
            pltpu.SemaphoreType.DMA,
            pltpu.SemaphoreType.DMA,
        ],
    )
    def _combine(out2_hbm, pos0_hbm, pos1_hbm, y_hbm, i0_v, i1_v, r0_v, r1_v,
                 sema0, sema1, semb0, semb1):
        wid = lax.axis_index("s") * NC + lax.axis_index("c")
        base = wid * _TPW
        pltpu.sync_copy(pos0_hbm.at[pl.ds(base, _TPW)], i0_v)
        pltpu.sync_copy(pos1_hbm.at[pl.ds(base, _TPW)], i1_v)

        def start(c):
            rs = pl.ds(c * _CCH, _CCH)
            s0, s1 = (sema0, sema1) if c % 2 == 0 else (semb0, semb1)
            cp0 = pltpu.async_copy(out2_hbm.at[i0_v.at[rs]], r0_v.at[rs], s0)
            cp1 = pltpu.async_copy(out2_hbm.at[i1_v.at[rs]], r1_v.at[rs], s1)
            return cp0, cp1

        inflight = start(0)
        for c in range(_NCH):
            nxt = start(c + 1) if c + 1 < _NCH else None
            inflight[0].wait()
            inflight[1].wait()

            def row(r, _):
                for j in range(D // 16):
                    sl = pl.ds(j * 16, 16)
                    plsc.addupdate(r0_v.at[r, sl], r1_v[r, sl])
                return _

            lax.fori_loop(c * _CCH, (c + 1) * _CCH, row, 0)
            inflight = nxt
        pltpu.sync_copy(r0_v, y_hbm.at[pl.ds(base, _TPW)])

    return _combine


def kernel(x, Wg, bg, W1, b1, W3, b3, W2, b2):
    (pos0, pos1, w0, w1, be, rs, par, nxe, hn, nreal) = _route(
        x, Wg.T, jnp.reshape(bg, (1, E)))
    xs, ws = _make_dispatch()(
        x, jnp.reshape(pos0, (T,)), jnp.reshape(pos1, (T,)), w0, w1)
    out2 = _ffn(jnp.reshape(be, (NBLK,)), jnp.reshape(rs, (NBLK,)),
                jnp.reshape(par, (NBLK,)), jnp.reshape(nxe, (NBLK,)),
                jnp.reshape(hn, (NBLK,)), jnp.reshape(nreal, (NBLK,)), xs,
                W1, jnp.reshape(b1, (E, 1, I)),
                W3, jnp.reshape(b3, (E, 1, I)),
                W2, jnp.reshape(b2, (E, 1, D)), ws)
    y = _make_combine()(out2, jnp.reshape(pos0, (T,)), jnp.reshape(pos1, (T,)))
    return y


# speedup vs baseline: 1.0464x; 1.0052x over previous
import functools

import jax
import jax.numpy as jnp
from jax import lax
from jax.experimental import pallas as pl
from jax.experimental.pallas import tpu as pltpu
from jax.experimental.pallas import tpu_sc as plsc

T, D, I, E, K = 2048, 768, 1024, 8, 2
D2 = D // 2
RB = 128
NSLOT = 5120
NBLK = NSLOT // RB
WREP = 128
NC, NS = 2, 16
NW = NC * NS
NEG = -1e30


def _fiota(shape, dim):
    return lax.broadcasted_iota(jnp.int32, shape, dim).astype(jnp.float32)


def _route_body(x_ref, wg_ref, bg_ref, pos0_ref, pos1_ref, w0_ref, w1_ref,
                be_ref, rs_ref, par_ref, nxe_ref, hn_ref, nreal_ref):
    logits = lax.dot_general(x_ref[...], wg_ref[...],
                             (((1,), (1,)), ((), ())),
                             preferred_element_type=jnp.float32)
    logits = logits + bg_ref[...]
    m = jnp.max(logits, axis=-1, keepdims=True)
    p = jnp.exp(logits - m)
    s = p / jnp.sum(p, axis=-1, keepdims=True)

    ei = _fiota((T, E), 1)
    m0 = jnp.max(s, axis=-1, keepdims=True)
    i0 = jnp.min(jnp.where(s >= m0, ei, float(E)), axis=-1, keepdims=True)
    s2 = jnp.where(ei == i0, NEG, s)
    m1 = jnp.max(s2, axis=-1, keepdims=True)
    i1 = jnp.min(jnp.where(s2 >= m1, ei, float(E)), axis=-1, keepdims=True)

    w0_ref[...] = jnp.broadcast_to(m0, (T, WREP))
    w1_ref[...] = jnp.broadcast_to(m1, (T, WREP))

    m0e = (ei == i0).astype(jnp.float32)
    m1e = (ei == i1).astype(jnp.float32)

    ri = _fiota((T, T), 0)
    ci = _fiota((T, T), 1)
    ltri = (ci <= ri).astype(jnp.bfloat16)
    mcat = jnp.concatenate([m0e, m1e], axis=1).astype(jnp.bfloat16)
    c01 = jnp.dot(ltri, mcat, preferred_element_type=jnp.float32)
    c0 = c01[:, :E]
    c1 = c01[:, E:]

    n0 = c0[T - 1:T, :]
    n1 = c1[T - 1:T, :]
    n = n0 + n1
    nblk = jnp.floor((n + (RB - 1.0)) * (1.0 / RB))
    r8 = _fiota((E, E), 0)
    c8 = _fiota((E, E), 1)
    sutri = (r8 < c8).astype(jnp.float32)
    blkoff = jnp.dot(nblk, sutri, preferred_element_type=jnp.float32)
    off = blkoff * RB

    ones8 = jnp.ones((1, E), jnp.float32)
    dnt = (((1,), (1,)), ((), ()))

    def _posrow(mask, q):
        qh = jnp.floor(q * (1.0 / RB))
        ql = q - RB * qh
        return (RB * lax.dot_general(ones8, mask * qh, dnt,
                                     preferred_element_type=jnp.float32)
                + lax.dot_general(ones8, mask * ql, dnt,
                                  preferred_element_type=jnp.float32))

    pos0 = _posrow(m0e, off + c0 - m0e)
    pos1 = _posrow(m1e, off + n0 + c1 - m1e)
    pos0_ref[...] = pos0.astype(jnp.int32)
    pos1_ref[...] = pos1.astype(jnp.int32)

    bi = _fiota((E, NBLK), 1)
    eb = _fiota((E, NBLK), 0)
    boffc = jnp.reshape(blkoff, (E, 1))
    nblkc = jnp.reshape(nblk, (E, 1))
    ind = ((bi >= boffc) & (bi < boffc + nblkc)).astype(jnp.float32)
    tail = ((eb == (E - 1.0)) &
            (jnp.sum(ind, axis=0, keepdims=True) == 0.0)).astype(jnp.float32)
    ind2 = jnp.minimum(ind + tail, 1.0)
    bex = jnp.sum(eb * ind2, axis=0, keepdims=True)
    be_ref[...] = bex.astype(jnp.int32)

    presentc = jnp.minimum(
        jnp.sum(ind, axis=1, keepdims=True).astype(jnp.bool_).astype(
            jnp.float32)
        + (_fiota((E, 1), 0) == (E - 1.0)).astype(jnp.float32), 1.0)
    ltm = (c8 < r8).astype(jnp.float32)
    rankc = jnp.dot(ltm, presentc, preferred_element_type=jnp.float32)
    parc = rankc - 2.0 * jnp.floor(rankc * 0.5)
    gtm = (c8 > r8).astype(jnp.float32)
    prow = jnp.sum((r8 == c8).astype(jnp.float32) * presentc, axis=0,
                   keepdims=True)
    candm = gtm * prow * c8 + (1.0 - gtm * prow) * 1e9
    nxt = jnp.min(candm, axis=1, keepdims=True)
    nxt = jnp.where(nxt > float(E), _fiota((E, 1), 0), nxt)
    par_b = jnp.sum(parc * ind2, axis=0, keepdims=True)
    nxe_b = jnp.sum(nxt * ind2, axis=0, keepdims=True)
    rs_b = (bex != jnp.concatenate([bex[:, :1] - 1.0, bex[:, :NBLK - 1]],
                                   axis=1)).astype(jnp.float32)
    hn_b = (nxe_b != bex).astype(jnp.float32)
    par_ref[...] = par_b.astype(jnp.int32)
    nxe_ref[...] = nxe_b.astype(jnp.int32)
    rs_ref[...] = rs_b.astype(jnp.int32)
    hn_ref[...] = hn_b.astype(jnp.int32)
    nreal_ref[...] = jnp.broadcast_to(
        jnp.sum(nblk, axis=1, keepdims=True), (1, NBLK)).astype(jnp.int32)


def _route(x, Wg, bg):
    return pl.pallas_call(
        _route_body,
        out_shape=(
            jax.ShapeDtypeStruct((1, T), jnp.int32),
            jax.ShapeDtypeStruct((1, T), jnp.int32),
            jax.ShapeDtypeStruct((T, WREP), jnp.float32),
            jax.ShapeDtypeStruct((T, WREP), jnp.float32),
            jax.ShapeDtypeStruct((1, NBLK), jnp.int32),
            jax.ShapeDtypeStruct((1, NBLK), jnp.int32),
            jax.ShapeDtypeStruct((1, NBLK), jnp.int32),
            jax.ShapeDtypeStruct((1, NBLK), jnp.int32),
            jax.ShapeDtypeStruct((1, NBLK), jnp.int32),
            jax.ShapeDtypeStruct((1, NBLK), jnp.int32),
        ),
    )(x, Wg, bg)


_TPW = T // NW
_NCH = 4
_CCH = _TPW // _NCH


@functools.cache
def _sc_mesh():
    return plsc.VectorSubcoreMesh(core_axis_name="c", subcore_axis_name="s",
                                  num_cores=NC, num_subcores=NS)


@functools.cache
def _make_dispatch():
    @functools.partial(
        pl.kernel,
        out_type=(
            jax.ShapeDtypeStruct((NSLOT, D), jnp.float32),
            jax.ShapeDtypeStruct((NSLOT, WREP), jnp.float32),
        ),
        mesh=_sc_mesh(),
        scratch_types=[
            pltpu.VMEM((_TPW,), jnp.int32),
            pltpu.VMEM((_TPW,), jnp.int32),
            pltpu.VMEM((_TPW, D), jnp.float32),
            pltpu.VMEM((_TPW, WREP), jnp.float32),
            pltpu.VMEM((_TPW, WREP), jnp.float32),
            pltpu.SemaphoreType.DMA,
            pltpu.SemaphoreType.DMA,
            pltpu.SemaphoreType.DMA,
            pltpu.SemaphoreType.DMA,
            pltpu.SemaphoreType.DMA,
        ],
    )
    def _dispatch(x_hbm, pos0_hbm, pos1_hbm, w0_hbm, w1_hbm, xs_hbm, ws_hbm,
                  i0_v, i1_v, rows_v, w0_v, w1_v, s1, s2, s3, s4, s5):
        wid = lax.axis_index("s") * NC + lax.axis_index("c")
        base = wid * _TPW
        sl = pl.ds(base, _TPW)
        ci0 = pltpu.async_copy(pos0_hbm.at[sl], i0_v, s1)
        ci1 = pltpu.async_copy(pos1_hbm.at[sl], i1_v, s2)
        cx = pltpu.async_copy(x_hbm.at[sl], rows_v, s3)
        cw0 = pltpu.async_copy(w0_hbm.at[sl], w0_v, s4)
        cw1 = pltpu.async_copy(w1_hbm.at[sl], w1_v, s5)
        ci0.wait()
        cx.wait()
        sc0 = pltpu.async_copy(rows_v, xs_hbm.at[i0_v], s1)
        ci1.wait()
        sc1 = pltpu.async_copy(rows_v, xs_hbm.at[i1_v], s2)
        cw0.wait()
        cw1.wait()
        sw0 = pltpu.async_copy(w0_v, ws_hbm.at[i0_v], s3)
        sw1 = pltpu.async_copy(w1_v, ws_hbm.at[i1_v], s4)
        sc0.wait()
        sc1.wait()
        sw0.wait()
        sw1.wait()

    return _dispatch


def _ffn_body(be_ref, rs_ref, par_ref, nxe_ref, hn_ref, nreal_ref,
              xs_ref, w1_hbm, b1_ref, w3_hbm, b3_ref, w2_hbm, b2_ref, ws_ref,
              out_ref, w1b_v, w3b_v, w2b_v, sem0, sem1):
    b = pl.program_id(0)
    e = be_ref[b]
    slot = par_ref[b]
    sems = [sem0, sem1]

    def cps(dste, s):
        sem = sems[0] if isinstance(s, int) and s == 0 else sems[1]
        return (
            pltpu.make_async_copy(w1_hbm.at[dste], w1b_v.at[s], sem),
            pltpu.make_async_copy(w3_hbm.at[dste], w3b_v.at[s], sem),
            pltpu.make_async_copy(w2_hbm.at[dste], w2b_v.at[s], sem),
        )

    @pl.when(b == 0)
    def _():
        for cp in cps(e, 0):
            cp.start()

    @pl.when(rs_ref[b] == 1)
    def _():
        @pl.when(hn_ref[b] == 1)
        def _():
            @pl.when(slot == 0)
            def _():
                for cp in cps(nxe_ref[b], 1):
                    cp.start()

            @pl.when(slot == 1)
            def _():
                for cp in cps(nxe_ref[b], 0):
                    cp.start()

        @pl.when(slot == 0)
        def _():
            for cp in cps(e, 0):
                cp.wait()

        @pl.when(slot == 1)
        def _():
            for cp in cps(e, 1):
                cp.wait()

    @pl.when(b < nreal_ref[0])
    def _():
        xsb = xs_ref[...].astype(jnp.bfloat16)
        w1b = w1b_v[slot].astype(jnp.bfloat16)
        w3b = w3b_v[slot].astype(jnp.bfloat16)
        g = jnp.dot(xsb, w1b, preferred_element_type=jnp.float32) + b1_ref[0]
        u = jnp.dot(xsb, w3b, preferred_element_type=jnp.float32) + b3_ref[0]
        h = g * (1.0 / (1.0 + jnp.exp(-g))) * u
        hb = h.astype(jnp.bfloat16)
        w2b = w2b_v[slot].astype(jnp.bfloat16)
        o = jnp.dot(hb, w2b, preferred_element_type=jnp.float32) + b2_ref[0]
        out_ref[...] = o * ws_ref[:, :1]


def _ffn(be, rs, par, nxe, hn, nreal, xs, W1, b1, W3, b3, W2, b2, ws):
    grid_spec = pltpu.PrefetchScalarGridSpec(
        num_scalar_prefetch=6,
        grid=(NBLK,),
        in_specs=[
            pl.BlockSpec((RB, D), lambda b, *_: (b, 0)),
            pl.BlockSpec(memory_space=pltpu.HBM),
            pl.BlockSpec((1, 1, I), lambda b, be_ref, *_: (be_ref[b], 0, 0)),
            pl.BlockSpec(memory_space=pltpu.HBM),
            pl.BlockSpec((1, 1, I), lambda b, be_ref, *_: (be_ref[b], 0, 0)),
            pl.BlockSpec(memory_space=pltpu.HBM),
            pl.BlockSpec((1, 1, D), lambda b, be_ref, *_: (be_ref[b], 0, 0)),
            pl.BlockSpec((RB, WREP), lambda b, *_: (b, 0)),
        ],
        out_specs=pl.BlockSpec((RB, D), lambda b, *_: (b, 0)),
        scratch_shapes=[
            pltpu.VMEM((2, D, I), jnp.float32),
            pltpu.VMEM((2, D, I), jnp.float32),
            pltpu.VMEM((2, I, D), jnp.float32),
            pltpu.SemaphoreType.DMA,
            pltpu.SemaphoreType.DMA,
        ],
    )
    return pl.pallas_call(
        _ffn_body,
        grid_spec=grid_spec,
        out_shape=jax.ShapeDtypeStruct((NSLOT, D), jnp.float32),
        compiler_params=pltpu.CompilerParams(
            dimension_semantics=("arbitrary",),
        ),
    )(be, rs, par, nxe, hn, nreal, xs, W1, b1, W3, b3, W2, b2, ws)


@functools.cache
def _make_combine():
    @functools.partial(
        pl.kernel,
        out_type=jax.ShapeDtypeStruct((T, D), jnp.float32),
        mesh=_sc_mesh(),
        scratch_types=[
            pltpu.VMEM((_TPW,), jnp.int32),
            pltpu.VMEM((_TPW,), jnp.int32),
            pltpu.VMEM((_TPW, D), jnp.float32),
            pltpu.VMEM((_TPW, D), jnp.float32),
            pltpu.SemaphoreType.DMA,
            pltpu.SemaphoreType.DMA,
            pltpu.SemaphoreType.DMA,
            pltpu.SemaphoreType.DMA,
        ],
    )
    def _combine(out2_hbm, pos0_hbm, pos1_hbm, y_hbm, i0_v, i1_v, r0_v, r1_v,
                 sema0, sema1, semb0, semb1):
        wid = lax.axis_index("s") * NC + lax.axis_index("c")
        base = wid * _TPW
        ci0 = pltpu.async_copy(pos0_hbm.at[pl.ds(base, _TPW)], i0_v, sema0)
        ci1 = pltpu.async_copy(pos1_hbm.at[pl.ds(base, _TPW)], i1_v, sema1)
        ci0.wait()
        ci1.wait()

        def start(c):
            rs = pl.ds(c * _CCH, _CCH)
            s0, s1 = (sema0, sema1) if c % 2 == 0 else (semb0, semb1)
            cp0 = pltpu.async_copy(out2_hbm.at[i0_v.at[rs]], r0_v.at[rs], s0)
            cp1 = pltpu.async_copy(out2_hbm.at[i1_v.at[rs]], r1_v.at[rs], s1)
            return cp0, cp1

        inflight = start(0)
        for c in range(_NCH):
            nxt = start(c + 1) if c + 1 < _NCH else None
            inflight[0].wait()
            inflight[1].wait()

            def row(r, _):
                for j in range(D // 16):
                    sl = pl.ds(j * 16, 16)
                    plsc.addupdate(r0_v.at[r, sl], r1_v[r, sl])
                return _

            lax.fori_loop(c * _CCH, (c + 1) * _CCH, row, 0)
            inflight = nxt
        pltpu.sync_copy(r0_v, y_hbm.at[pl.ds(base, _TPW)])

    return _combine


def kernel(x, Wg, bg, W1, b1, W3, b3, W2, b2):
    (pos0, pos1, w0, w1, be, rs, par, nxe, hn, nreal) = _route(
        x, Wg.T, jnp.reshape(bg, (1, E)))
    xs, ws = _make_dispatch()(
        x, jnp.reshape(pos0, (T,)), jnp.reshape(pos1, (T,)), w0, w1)
    out2 = _ffn(jnp.reshape(be, (NBLK,)), jnp.reshape(rs, (NBLK,)),
                jnp.reshape(par, (NBLK,)), jnp.reshape(nxe, (NBLK,)),
                jnp.reshape(hn, (NBLK,)), jnp.reshape(nreal, (NBLK,)), xs,
                W1, jnp.reshape(b1, (E, 1, I)),
                W3, jnp.reshape(b3, (E, 1, I)),
                W2, jnp.reshape(b2, (E, 1, D)), ws)
    y = _make_combine()(out2, jnp.reshape(pos0, (T,)), jnp.reshape(pos1, (T,)))
    return y
